# baseline (device time: 35199 ns/iter reference)
import jax
import jax.numpy as jnp
from jax import lax
from jax.experimental import pallas as pl
from jax.experimental.pallas import tpu as pltpu

ROWS = 256
K = 512
HALF = 4096
NC = 8
CHUNK = HALF // NC


def kernel(x, W):
    def body(
        x_hbm,
        w_hbm,
        out_hbm,
        x_vmem,
        w_vmem,
        send_buf,
        recv_buf,
        x_sem,
        w_sems,
        out_sems,
        send_sems,
        recv_sems,
    ):
        my_x = lax.axis_index("x")
        my_y = lax.axis_index("y")
        my_z = lax.axis_index("z")
        partner = (my_x, 1 - my_y, my_z)

        x_copy = pltpu.make_async_copy(x_hbm, x_vmem, x_sem)
        x_copy.start()

        def w_copy(k):
            return pltpu.make_async_copy(
                w_hbm.at[:, pl.ds(k * CHUNK, CHUNK)],
                w_vmem.at[k % 2],
                w_sems.at[k % 2],
            )

        w_copy(0).start()

        barrier_sem = pltpu.get_barrier_semaphore()
        pl.semaphore_signal(
            barrier_sem, inc=1, device_id=partner,
            device_id_type=pl.DeviceIdType.MESH,
        )
        pl.semaphore_wait(barrier_sem, 1)

        x_copy.wait()
        xl = x_vmem[...].astype(jnp.bfloat16)

        def chunk_rdma(k):
            cs = pl.ds(k * CHUNK, CHUNK)
            return pltpu.make_async_remote_copy(
                src_ref=send_buf.at[:, cs],
                dst_ref=recv_buf.at[:, cs],
                send_sem=send_sems.at[k],
                recv_sem=recv_sems.at[k],
                device_id=partner,
                device_id_type=pl.DeviceIdType.MESH,
            )

        s_loc = jnp.zeros((ROWS, 1), jnp.float32)
        for k in range(NC):
            w_copy(k).wait()
            if k + 1 < NC:
                w_copy(k + 1).start()
            wk = w_vmem[k % 2].astype(jnp.bfloat16)
            ek = jnp.exp(jnp.dot(xl, wk, preferred_element_type=jnp.float32))
            cs = pl.ds(k * CHUNK, CHUNK)
            send_buf[:, cs] = ek.astype(jnp.bfloat16)
            chunk_rdma(k).start()
            s_loc = s_loc + jnp.sum(ek, axis=1, keepdims=True)

        s_rem = jnp.zeros((ROWS, 1), jnp.float32)
        for k in range(NC):
            chunk_rdma(k).wait_recv()
            ck = recv_buf[:, pl.ds(k * CHUNK, CHUNK)].astype(jnp.float32)
            s_rem = s_rem + jnp.sum(ck, axis=1, keepdims=True)

        for k in range(NC):
            chunk_rdma(k).wait_send()

        inv = 1.0 / (s_loc + s_rem)
        loc_off = my_y * HALF
        rem_off = (1 - my_y) * HALF

        send_buf[...] = (send_buf[...].astype(jnp.float32) * inv).astype(
            jnp.bfloat16
        )
        out_loc = pltpu.make_async_copy(
            send_buf, out_hbm.at[:, pl.ds(loc_off, HALF)], out_sems.at[0]
        )
        out_loc.start()

        recv_buf[...] = (recv_buf[...].astype(jnp.float32) * inv).astype(
            jnp.bfloat16
        )
        out_rem = pltpu.make_async_copy(
            recv_buf, out_hbm.at[:, pl.ds(rem_off, HALF)], out_sems.at[1]
        )
        out_rem.start()

        out_loc.wait()
        out_rem.wait()

    return pl.pallas_call(
        body,
        out_shape=jax.ShapeDtypeStruct((ROWS, 2 * HALF), jnp.bfloat16),
        in_specs=[
            pl.BlockSpec(memory_space=pl.ANY),
            pl.BlockSpec(memory_space=pl.ANY),
        ],
        out_specs=pl.BlockSpec(memory_space=pl.ANY),
        scratch_shapes=[
            pltpu.VMEM((ROWS, K), jnp.float32),
            pltpu.VMEM((2, K, CHUNK), jnp.float32),
            pltpu.VMEM((ROWS, HALF), jnp.bfloat16),
            pltpu.VMEM((ROWS, HALF), jnp.bfloat16),
            pltpu.SemaphoreType.DMA,
            pltpu.SemaphoreType.DMA((2,)),
            pltpu.SemaphoreType.DMA((2,)),
            pltpu.SemaphoreType.DMA((NC,)),
            pltpu.SemaphoreType.DMA((NC,)),
        ],
        compiler_params=pltpu.CompilerParams(collective_id=0),
    )(x, W)


# device time: 31590 ns/iter; 1.1142x vs baseline; 1.1142x over previous
import jax
import jax.numpy as jnp
from jax import lax
from jax.experimental import pallas as pl
from jax.experimental.pallas import tpu as pltpu

ROWS = 256
K = 512
HALF = 4096
NC = 8
CHUNK = HALF // NC


def kernel(x, W):
    def body(
        x_hbm,
        w_hbm,
        out_hbm,
        x_vmem,
        w_vmem,
        send_buf,
        recv_buf,
        x_sem,
        w_sems,
        out_sems,
        send_sems,
        recv_sems,
    ):
        my_x = lax.axis_index("x")
        my_y = lax.axis_index("y")
        my_z = lax.axis_index("z")
        partner = (my_x, 1 - my_y, my_z)

        x_copy = pltpu.make_async_copy(x_hbm, x_vmem, x_sem)
        x_copy.start()

        def w_copy(k):
            return pltpu.make_async_copy(
                w_hbm.at[:, pl.ds(k * CHUNK, CHUNK)],
                w_vmem.at[k % 2],
                w_sems.at[k % 2],
            )

        w_copy(0).start()

        barrier_sem = pltpu.get_barrier_semaphore()
        pl.semaphore_signal(
            barrier_sem, inc=1, device_id=partner,
            device_id_type=pl.DeviceIdType.MESH,
        )
        pl.semaphore_wait(barrier_sem, 1)

        x_copy.wait()
        xl = x_vmem[...].astype(jnp.bfloat16)

        def chunk_rdma(k):
            cs = pl.ds(k * CHUNK, CHUNK)
            return pltpu.make_async_remote_copy(
                src_ref=send_buf.at[:, cs],
                dst_ref=recv_buf.at[:, cs],
                send_sem=send_sems.at[k],
                recv_sem=recv_sems.at[k],
                device_id=partner,
                device_id_type=pl.DeviceIdType.MESH,
            )

        s_loc = jnp.zeros((ROWS, 1), jnp.float32)
        for k in range(NC):
            w_copy(k).wait()
            if k + 1 < NC:
                w_copy(k + 1).start()
            wk = w_vmem[k % 2].astype(jnp.bfloat16)
            ek = jnp.exp(jnp.dot(xl, wk, preferred_element_type=jnp.float32))
            cs = pl.ds(k * CHUNK, CHUNK)
            send_buf[:, cs] = ek.astype(jnp.bfloat16)
            chunk_rdma(k).start()
            s_loc = s_loc + jnp.sum(ek, axis=1, keepdims=True)

        s_rem = jnp.zeros((ROWS, 1), jnp.float32)
        for k in range(NC):
            chunk_rdma(k).wait_recv()
            ck = recv_buf[:, pl.ds(k * CHUNK, CHUNK)].astype(jnp.float32)
            s_rem = s_rem + jnp.sum(ck, axis=1, keepdims=True)

        for k in range(NC):
            chunk_rdma(k).wait_send()

        inv = 1.0 / (s_loc + s_rem)
        loc_off = my_y * HALF
        rem_off = (1 - my_y) * HALF

        send_buf[...] = (send_buf[...].astype(jnp.float32) * inv).astype(
            jnp.bfloat16
        )
        out_loc = pltpu.make_async_copy(
            send_buf, out_hbm.at[:, pl.ds(loc_off, HALF)], out_sems.at[0]
        )
        out_loc.start()

        recv_buf[...] = (recv_buf[...].astype(jnp.float32) * inv).astype(
            jnp.bfloat16
        )
        out_rem = pltpu.make_async_copy(
            recv_buf, out_hbm.at[:, pl.ds(rem_off, HALF)], out_sems.at[1]
        )
        out_rem.start()

        out_loc.wait()
        out_rem.wait()

    return pl.pallas_call(
        body,
        out_shape=jax.ShapeDtypeStruct((ROWS, 2 * HALF), jnp.bfloat16),
        in_specs=[
            pl.BlockSpec(memory_space=pltpu.MemorySpace.HBM),
            pl.BlockSpec(memory_space=pltpu.MemorySpace.HBM),
        ],
        out_specs=pl.BlockSpec(memory_space=pl.ANY),
        scratch_shapes=[
            pltpu.VMEM((ROWS, K), jnp.float32),
            pltpu.VMEM((2, K, CHUNK), jnp.float32),
            pltpu.VMEM((ROWS, HALF), jnp.bfloat16),
            pltpu.VMEM((ROWS, HALF), jnp.bfloat16),
            pltpu.SemaphoreType.DMA,
            pltpu.SemaphoreType.DMA((2,)),
            pltpu.SemaphoreType.DMA((2,)),
            pltpu.SemaphoreType.DMA((NC,)),
            pltpu.SemaphoreType.DMA((NC,)),
        ],
        compiler_params=pltpu.CompilerParams(collective_id=0),
    )(
        pltpu.with_memory_space_constraint(x, pltpu.MemorySpace.HBM),
        pltpu.with_memory_space_constraint(W, pltpu.MemorySpace.HBM),
    )
